# Initial kernel scaffold; baseline (speedup 1.0000x reference)
import jax
import jax.numpy as jnp
from jax.experimental import pallas as pl


def kernel(x, edge_index, bn_w, bn_b, W1, b1, W2, b2):
    raise NotImplementedError("write your pallas kernel here")


# speedup vs baseline: 11.9832x; 11.9832x over previous
import functools

import jax
import jax.numpy as jnp
from jax import lax
from jax.experimental import pallas as pl
from jax.experimental.pallas import tpu as pltpu
from jax.experimental.pallas import tpu_sc as plsc

N = 10000
E = 160000
IN_DIM = 256
H_DIM = 512
OUT_DIM = 256
HALF = 128

NC = 2
NS = 16
K = 128

PROP_CHUNKS = 80
EP = NS * PROP_CHUNKS * K
DEG_CHUNKS = 40
ACC_ROWS = 10240
ZROWS = 16
OUT_RPT = N // NS

_mesh = plsc.VectorSubcoreMesh(core_axis_name="c", subcore_axis_name="s")


def _sc_degree_body(dst_hbm, out_hbm, dstv, onev, zbuf, hist, sem):
    c = lax.axis_index("c")
    s = lax.axis_index("s")
    for i in range(K // 16):
        onev[pl.ds(i * 16, 16)] = jnp.ones((16,), jnp.float32)
    for i in range(640 // 16):
        zbuf[pl.ds(i * 16, 16)] = jnp.zeros((16,), jnp.float32)
    pltpu.sync_copy(zbuf, hist.at[pl.ds(s * 640, 640)])
    plsc.subcore_barrier()

    def chunk(j, carry):
        base = c * (EP // 2) + s * (DEG_CHUNKS * K) + j * K
        pltpu.sync_copy(dst_hbm.at[pl.ds(base, K)], dstv)
        pltpu.sync_copy(onev, hist.at[dstv], add=True)
        return carry

    lax.fori_loop(0, DEG_CHUNKS, chunk, 0)
    plsc.subcore_barrier()
    pltpu.sync_copy(hist.at[pl.ds(s * 640, 640)],
                    out_hbm.at[c, pl.ds(s * 640, 640)])


_sc_degree = pl.kernel(
    _sc_degree_body,
    out_type=jax.ShapeDtypeStruct((NC, ACC_ROWS), jnp.float32),
    mesh=_mesh,
    scratch_types=[
        pltpu.VMEM((K,), jnp.int32),
        pltpu.VMEM((K,), jnp.float32),
        pltpu.VMEM((640,), jnp.float32),
        pltpu.VMEM_SHARED((ACC_ROWS,), jnp.float32),
        pltpu.SemaphoreType.DMA,
    ],
)


def _sc_prop_body(ya, yb, src_hbm, dst_hbm, out_hbm,
                  srcv, dstv, rows, zrow, acc, sem):
    c = lax.axis_index("c")
    s = lax.axis_index("s")
    for r in range(ZROWS):
        for q in range(HALF // 16):
            zrow[r, pl.ds(q * 16, 16)] = jnp.zeros((16,), jnp.float32)

    def zchunk(j, carry):
        pltpu.sync_copy(zrow, acc.at[pl.ds(s * 640 + j * ZROWS, ZROWS)])
        return carry

    lax.fori_loop(0, 640 // ZROWS, zchunk, 0)
    plsc.subcore_barrier()

    def chunk(j, carry):
        base = s * (PROP_CHUNKS * K) + j * K
        pltpu.sync_copy(src_hbm.at[pl.ds(base, K)], srcv)
        pltpu.sync_copy(dst_hbm.at[pl.ds(base, K)], dstv)

        @pl.when(c == 0)
        def _():
            pltpu.async_copy(ya.at[srcv], rows, sem).wait()

        @pl.when(c == 1)
        def _():
            pltpu.async_copy(yb.at[srcv], rows, sem).wait()

        pltpu.sync_copy(rows, acc.at[dstv], add=True)
        return carry

    lax.fori_loop(0, PROP_CHUNKS, chunk, 0)
    plsc.subcore_barrier()
    pltpu.sync_copy(acc.at[pl.ds(s * 640, 640)],
                    out_hbm.at[c, pl.ds(s * 640, 640)])


_sc_prop = pl.kernel(
    _sc_prop_body,
    out_type=jax.ShapeDtypeStruct((NC, ACC_ROWS, HALF), jnp.float32),
    mesh=_mesh,
    scratch_types=[
        pltpu.VMEM((K,), jnp.int32),
        pltpu.VMEM((K,), jnp.int32),
        pltpu.VMEM((K, HALF), jnp.float32),
        pltpu.VMEM((ZROWS, HALF), jnp.float32),
        pltpu.VMEM_SHARED((ACC_ROWS, HALF), jnp.float32),
        pltpu.SemaphoreType.DMA,
    ],
)


def _tc_prep_body(x_ref, bnw_ref, bnb_ref, degs_ref,
                  y1_ref, u1_ref, dinv_ref, invd_ref):
    x = x_ref[...]
    mean = jnp.mean(x, axis=0)
    xc = x - mean[None, :]
    var = jnp.mean(xc * xc, axis=0)
    t = xc * lax.rsqrt(var + 1e-5)[None, :] * bnw_ref[...][None, :] \
        + bnb_ref[...][None, :]
    deg = degs_ref[0, :N] + degs_ref[1, :N] + 1.0
    invd = 1.0 / deg
    dinv = lax.rsqrt(deg)
    y1 = t * dinv[:, None]
    y1_ref[0] = y1[:, :HALF]
    y1_ref[1] = y1[:, HALF:]
    u1_ref[...] = t * invd[:, None]
    dinv_ref[...] = dinv[:, None]
    invd_ref[...] = invd[:, None]


def _tc_prep(x, bn_w, bn_b, degs):
    return pl.pallas_call(
        _tc_prep_body,
        out_shape=[
            jax.ShapeDtypeStruct((NC, N, HALF), jnp.float32),
            jax.ShapeDtypeStruct((N, IN_DIM), jnp.float32),
            jax.ShapeDtypeStruct((N, 1), jnp.float32),
            jax.ShapeDtypeStruct((N, 1), jnp.float32),
        ],
    )(x, bn_w, bn_b, degs)


BM = 1000


def _tc_mid_body(z_ref, u1_ref, dinv_ref, invd_ref, W1_ref, b1_ref, W2_ref,
                 y2_ref, u2_ref):
    z = jnp.concatenate([z_ref[0], z_ref[1]], axis=1)
    p1 = dinv_ref[...] * z + u1_ref[...]
    h = jnp.dot(p1, W1_ref[...], preferred_element_type=jnp.float32)
    h = jnp.maximum(h + b1_ref[...][None, :], 0.0)
    t2 = jnp.dot(h, W2_ref[...], preferred_element_type=jnp.float32)
    y2 = t2 * dinv_ref[...]
    y2_ref[0] = y2[:, :HALF]
    y2_ref[1] = y2[:, HALF:]
    u2_ref[...] = t2 * invd_ref[...]


def _tc_mid(z1, u1, dinv, invd, W1, b1, W2):
    nb = N // BM
    return pl.pallas_call(
        _tc_mid_body,
        grid=(nb,),
        in_specs=[
            pl.BlockSpec((NC, BM, HALF), lambda i: (0, i, 0)),
            pl.BlockSpec((BM, IN_DIM), lambda i: (i, 0)),
            pl.BlockSpec((BM, 1), lambda i: (i, 0)),
            pl.BlockSpec((BM, 1), lambda i: (i, 0)),
            pl.BlockSpec((IN_DIM, H_DIM), lambda i: (0, 0)),
            pl.BlockSpec((H_DIM,), lambda i: (0,)),
            pl.BlockSpec((H_DIM, OUT_DIM), lambda i: (0, 0)),
        ],
        out_specs=[
            pl.BlockSpec((NC, BM, HALF), lambda i: (0, i, 0)),
            pl.BlockSpec((BM, OUT_DIM), lambda i: (i, 0)),
        ],
        out_shape=[
            jax.ShapeDtypeStruct((NC, N, HALF), jnp.float32),
            jax.ShapeDtypeStruct((N, OUT_DIM), jnp.float32),
        ],
    )(z1, u1, dinv, invd, W1, b1, W2)


def _tc_out_body(z_ref, u2_ref, dinv_ref, b2_ref, out_ref):
    z = jnp.concatenate([z_ref[0], z_ref[1]], axis=1)
    out_ref[...] = dinv_ref[...] * z + u2_ref[...] + b2_ref[...][None, :]


def _tc_out(z2, u2, dinv, b2):
    nb = N // BM
    return pl.pallas_call(
        _tc_out_body,
        grid=(nb,),
        in_specs=[
            pl.BlockSpec((NC, BM, HALF), lambda i: (0, i, 0)),
            pl.BlockSpec((BM, OUT_DIM), lambda i: (i, 0)),
            pl.BlockSpec((BM, 1), lambda i: (i, 0)),
            pl.BlockSpec((OUT_DIM,), lambda i: (0,)),
        ],
        out_specs=pl.BlockSpec((BM, OUT_DIM), lambda i: (i, 0)),
        out_shape=jax.ShapeDtypeStruct((N, OUT_DIM), jnp.float32),
    )(z2, u2, dinv, b2)


def kernel(x, edge_index, bn_w, bn_b, W1, b1, W2, b2):
    src = edge_index[0].astype(jnp.int32)
    dst = edge_index[1].astype(jnp.int32)
    npad = EP - E
    pad_ar = jnp.arange(npad, dtype=jnp.int32)
    src_p = jnp.concatenate([src, pad_ar % N])
    dst_p = jnp.concatenate([dst, N + (pad_ar % 16)])

    degs = _sc_degree(dst_p)
    y1, u1, dinv, invd = _tc_prep(x, bn_w, bn_b, degs)
    z1 = _sc_prop(y1[0], y1[1], src_p, dst_p)
    y2, u2 = _tc_mid(z1, u1, dinv, invd, W1, b1, W2)
    z2 = _sc_prop(y2[0], y2[1], src_p, dst_p)
    return _tc_out(z2, u2, dinv, b2)
